# baseline (speedup 1.0000x reference)
import jax
import jax.numpy as jnp
from jax.experimental import pallas as pl
from jax.experimental.pallas import tpu as pltpu

_NUM_EXPERTS = 8
_HIDDEN = 1024
_FFN = 4096
_TOKENS = 4096
_PACKED = _HIDDEN // 2

_B_T = 128
_B_F = 4096
_T_BLOCKS = _TOKENS // _B_T
_F_BLOCKS = _FFN // _B_F
_G = _T_BLOCKS + _NUM_EXPERTS - 1


def _moe_body(meta_ref, x_ref, pw_ref, sc_ref, zpsc_ref, rs_ref, out_ref, q_ref):
    i = pl.program_id(1)
    blk = meta_ref[0, i]
    start = meta_ref[1, i]
    end = meta_ref[2, i]
    prev = jnp.where(i > 0, i - 1, 0)
    first = jnp.logical_or(i == 0, blk != meta_ref[0, prev])
    need_unpack = jnp.logical_or(i == 0, meta_ref[3, i] != meta_ref[3, prev])

    @pl.when(need_unpack)
    def _():
        pw = pw_ref[0].astype(jnp.int32)
        q = jnp.concatenate([pw & 15, pw >> 4], axis=1).astype(jnp.float32)
        q_ref[...] = (q * sc_ref[0, 0][:, None]).astype(jnp.bfloat16)

    dot = jax.lax.dot_general(
        x_ref[...], q_ref[...], (((1,), (1,)), ((), ())),
        preferred_element_type=jnp.float32,
    )

    tok = blk * _B_T + jax.lax.broadcasted_iota(jnp.int32, (_B_T, 1), 0)
    mask = jnp.logical_and(tok >= start, tok < end)
    contrib = jnp.where(mask, dot - rs_ref[...] * zpsc_ref[0, 0][None, :], 0.0)

    @pl.when(first)
    def _():
        out_ref[...] = contrib

    @pl.when(jnp.logical_not(first))
    def _():
        out_ref[...] = out_ref[...] + contrib


def _permute_body(x_ref, p_ref, o_ref, rs_ref):
    x = x_ref[...]
    o_ref[...] = jax.lax.dot_general(
        x.astype(jnp.bfloat16), p_ref[...],
        (((1,), (0,)), ((), ())), preferred_element_type=jnp.float32,
    ).astype(jnp.bfloat16)
    rs_ref[...] = jnp.sum(x, axis=1, keepdims=True)


def _permute_x(inputs):
    col = jnp.arange(_HIDDEN, dtype=jnp.int32)
    src = jnp.where(col < _PACKED, col * 2, (col - _PACKED) * 2 + 1)
    perm = (col[:, None] == src[None, :]).astype(jnp.bfloat16)
    return pl.pallas_call(
        _permute_body,
        grid=(_T_BLOCKS,),
        in_specs=[
            pl.BlockSpec((_B_T, _HIDDEN), lambda i: (i, 0)),
            pl.BlockSpec((_HIDDEN, _HIDDEN), lambda i: (0, 0)),
        ],
        out_specs=[
            pl.BlockSpec((_B_T, _HIDDEN), lambda i: (i, 0)),
            pl.BlockSpec((_B_T, 1), lambda i: (i, 0)),
        ],
        out_shape=[
            jax.ShapeDtypeStruct((_TOKENS, _HIDDEN), jnp.bfloat16),
            jax.ShapeDtypeStruct((_TOKENS, 1), jnp.float32),
        ],
    )(inputs, perm)


def _build_plan(input_offsets, tokens_per_expert):
    starts = input_offsets.astype(jnp.int32)
    ends = starts + tokens_per_expert.astype(jnp.int32)
    blk_lo = jnp.arange(_T_BLOCKS, dtype=jnp.int32) * _B_T
    ov = jnp.logical_and(
        starts[None, :] < blk_lo[:, None] + _B_T, ends[None, :] > blk_lo[:, None]
    )
    flat = ov.reshape(-1)
    order = jnp.argsort(jnp.logical_not(flat), stable=True)[:_G]
    valid = flat[order]
    blk_ids = (order // _NUM_EXPERTS).astype(jnp.int32)
    e_ids = (order % _NUM_EXPERTS).astype(jnp.int32)
    n_real = jnp.sum(flat.astype(jnp.int32))
    last = jnp.maximum(n_real - 1, 0)
    blk_ids = jnp.where(valid, blk_ids, blk_ids[last])
    e_ids = jnp.where(valid, e_ids, e_ids[last])
    st = jnp.where(valid, starts[e_ids], 0)
    en = jnp.where(valid, ends[e_ids], 0)
    return jnp.stack([blk_ids, st, en, e_ids]).astype(jnp.int32)


def kernel(inputs, packed_weights, scales, zero_points, expert_ids,
           tokens_per_expert, input_offsets):
    del expert_ids
    meta = _build_plan(input_offsets, tokens_per_expert)
    x_perm, rowsums = _permute_x(inputs)
    sc3 = scales.reshape(_NUM_EXPERTS * _F_BLOCKS, 1, _B_F)
    zpsc3 = (zero_points * scales).reshape(_NUM_EXPERTS * _F_BLOCKS, 1, _B_F)

    grid_spec = pltpu.PrefetchScalarGridSpec(
        num_scalar_prefetch=1,
        grid=(_F_BLOCKS, _G),
        in_specs=[
            pl.BlockSpec((_B_T, _HIDDEN), lambda f, i, m: (m[0, i], 0)),
            pl.BlockSpec((1, _B_F, _PACKED), lambda f, i, m: (m[3, i], f, 0)),
            pl.BlockSpec((1, 1, _B_F), lambda f, i, m: (m[3, i] * _F_BLOCKS + f, 0, 0)),
            pl.BlockSpec((1, 1, _B_F), lambda f, i, m: (m[3, i] * _F_BLOCKS + f, 0, 0)),
            pl.BlockSpec((_B_T, 1), lambda f, i, m: (m[0, i], 0)),
        ],
        out_specs=pl.BlockSpec((_B_T, _B_F), lambda f, i, m: (m[0, i], f)),
        scratch_shapes=[pltpu.VMEM((_B_F, _HIDDEN), jnp.bfloat16)],
    )
    out = pl.pallas_call(
        _moe_body,
        grid_spec=grid_spec,
        out_shape=jax.ShapeDtypeStruct((_TOKENS, _FFN), jnp.float32),
        compiler_params=pltpu.CompilerParams(
            dimension_semantics=("arbitrary", "arbitrary"),
        ),
    )(meta, x_perm, packed_weights, sc3, zpsc3, rowsums)
    return out


# speedup vs baseline: 1.4500x; 1.4500x over previous
import jax
import jax.numpy as jnp
from jax.experimental import pallas as pl
from jax.experimental.pallas import tpu as pltpu

_NUM_EXPERTS = 8
_HIDDEN = 1024
_FFN = 4096
_TOKENS = 4096
_PACKED = _HIDDEN // 2

_B_T = 512
_B_F = 4096
_T_BLOCKS = _TOKENS // _B_T
_F_BLOCKS = _FFN // _B_F
_G = _T_BLOCKS + _NUM_EXPERTS - 1


def _moe_body(meta_ref, x_ref, pw_ref, sc_ref, zpsc_ref, rs_ref, out_ref, q_ref):
    i = pl.program_id(1)
    blk = meta_ref[0, i]
    start = meta_ref[1, i]
    end = meta_ref[2, i]
    prev = jnp.where(i > 0, i - 1, 0)
    first = jnp.logical_or(i == 0, blk != meta_ref[0, prev])
    need_unpack = jnp.logical_or(i == 0, meta_ref[3, i] != meta_ref[3, prev])

    @pl.when(need_unpack)
    def _():
        pw = pw_ref[0].astype(jnp.int32)
        q = jnp.concatenate([pw & 15, pw >> 4], axis=1).astype(jnp.float32)
        q_ref[...] = (q * sc_ref[0, 0][:, None]).astype(jnp.bfloat16)

    dot = jax.lax.dot_general(
        x_ref[...], q_ref[...], (((1,), (1,)), ((), ())),
        preferred_element_type=jnp.float32,
    )

    tok = blk * _B_T + jax.lax.broadcasted_iota(jnp.int32, (_B_T, 1), 0)
    mask = jnp.logical_and(tok >= start, tok < end)
    contrib = jnp.where(mask, dot - rs_ref[...] * zpsc_ref[0, 0][None, :], 0.0)

    @pl.when(first)
    def _():
        out_ref[...] = contrib

    @pl.when(jnp.logical_not(first))
    def _():
        out_ref[...] = out_ref[...] + contrib


def _permute_body(x_ref, p_ref, o_ref, rs_ref):
    x = x_ref[...]
    o_ref[...] = jax.lax.dot_general(
        x.astype(jnp.bfloat16), p_ref[...],
        (((1,), (0,)), ((), ())), preferred_element_type=jnp.float32,
    ).astype(jnp.bfloat16)
    rs_ref[...] = jnp.sum(x, axis=1, keepdims=True)


def _permute_x(inputs):
    col = jnp.arange(_HIDDEN, dtype=jnp.int32)
    src = jnp.where(col < _PACKED, col * 2, (col - _PACKED) * 2 + 1)
    perm = (col[:, None] == src[None, :]).astype(jnp.bfloat16)
    return pl.pallas_call(
        _permute_body,
        grid=(_T_BLOCKS,),
        in_specs=[
            pl.BlockSpec((_B_T, _HIDDEN), lambda i: (i, 0)),
            pl.BlockSpec((_HIDDEN, _HIDDEN), lambda i: (0, 0)),
        ],
        out_specs=[
            pl.BlockSpec((_B_T, _HIDDEN), lambda i: (i, 0)),
            pl.BlockSpec((_B_T, 1), lambda i: (i, 0)),
        ],
        out_shape=[
            jax.ShapeDtypeStruct((_TOKENS, _HIDDEN), jnp.bfloat16),
            jax.ShapeDtypeStruct((_TOKENS, 1), jnp.float32),
        ],
    )(inputs, perm)


def _build_plan(input_offsets, tokens_per_expert):
    starts = input_offsets.astype(jnp.int32)
    ends = starts + tokens_per_expert.astype(jnp.int32)
    blk_lo = jnp.arange(_T_BLOCKS, dtype=jnp.int32) * _B_T
    ov = jnp.logical_and(
        starts[None, :] < blk_lo[:, None] + _B_T, ends[None, :] > blk_lo[:, None]
    )
    flat = ov.reshape(-1)
    order = jnp.argsort(jnp.logical_not(flat), stable=True)[:_G]
    valid = flat[order]
    blk_ids = (order // _NUM_EXPERTS).astype(jnp.int32)
    e_ids = (order % _NUM_EXPERTS).astype(jnp.int32)
    n_real = jnp.sum(flat.astype(jnp.int32))
    last = jnp.maximum(n_real - 1, 0)
    blk_ids = jnp.where(valid, blk_ids, blk_ids[last])
    e_ids = jnp.where(valid, e_ids, e_ids[last])
    st = jnp.where(valid, starts[e_ids], 0)
    en = jnp.where(valid, ends[e_ids], 0)
    return jnp.stack([blk_ids, st, en, e_ids]).astype(jnp.int32)


def kernel(inputs, packed_weights, scales, zero_points, expert_ids,
           tokens_per_expert, input_offsets):
    del expert_ids
    meta = _build_plan(input_offsets, tokens_per_expert)
    x_perm, rowsums = _permute_x(inputs)
    sc3 = scales.reshape(_NUM_EXPERTS * _F_BLOCKS, 1, _B_F)
    zpsc3 = (zero_points * scales).reshape(_NUM_EXPERTS * _F_BLOCKS, 1, _B_F)

    grid_spec = pltpu.PrefetchScalarGridSpec(
        num_scalar_prefetch=1,
        grid=(_F_BLOCKS, _G),
        in_specs=[
            pl.BlockSpec((_B_T, _HIDDEN), lambda f, i, m: (m[0, i], 0)),
            pl.BlockSpec((1, _B_F, _PACKED), lambda f, i, m: (m[3, i], f, 0)),
            pl.BlockSpec((1, 1, _B_F), lambda f, i, m: (m[3, i] * _F_BLOCKS + f, 0, 0)),
            pl.BlockSpec((1, 1, _B_F), lambda f, i, m: (m[3, i] * _F_BLOCKS + f, 0, 0)),
            pl.BlockSpec((_B_T, 1), lambda f, i, m: (m[0, i], 0)),
        ],
        out_specs=pl.BlockSpec((_B_T, _B_F), lambda f, i, m: (m[0, i], f)),
        scratch_shapes=[pltpu.VMEM((_B_F, _HIDDEN), jnp.bfloat16)],
    )
    out = pl.pallas_call(
        _moe_body,
        grid_spec=grid_spec,
        out_shape=jax.ShapeDtypeStruct((_TOKENS, _FFN), jnp.float32),
        compiler_params=pltpu.CompilerParams(
            dimension_semantics=("arbitrary", "arbitrary"),
        ),
    )(meta, x_perm, packed_weights, sc3, zpsc3, rowsums)
    return out


# speedup vs baseline: 1.5668x; 1.0805x over previous
import jax
import jax.numpy as jnp
from jax.experimental import pallas as pl
from jax.experimental.pallas import tpu as pltpu

_NUM_EXPERTS = 8
_HIDDEN = 1024
_FFN = 4096
_TOKENS = 4096
_PACKED = _HIDDEN // 2

_B_T = 256
_B_F = 4096
_T_BLOCKS = _TOKENS // _B_T
_F_BLOCKS = _FFN // _B_F
_G = _T_BLOCKS + _NUM_EXPERTS - 1


def _moe_body(meta_ref, x_ref, pw_ref, sc_ref, zpsc_ref, rs_ref, out_ref, q_ref):
    i = pl.program_id(1)
    blk = meta_ref[0, i]
    start = meta_ref[1, i]
    end = meta_ref[2, i]
    prev = jnp.where(i > 0, i - 1, 0)
    first = jnp.logical_or(i == 0, blk != meta_ref[0, prev])
    need_unpack = jnp.logical_or(i == 0, meta_ref[3, i] != meta_ref[3, prev])

    @pl.when(need_unpack)
    def _():
        pw = pw_ref[0].astype(jnp.int32)
        q = jnp.concatenate([pw & 15, pw >> 4], axis=1).astype(jnp.float32)
        q_ref[...] = (q * sc_ref[0, 0][:, None]).astype(jnp.bfloat16)

    dot = jax.lax.dot_general(
        x_ref[...], q_ref[...], (((1,), (1,)), ((), ())),
        preferred_element_type=jnp.float32,
    )

    tok = blk * _B_T + jax.lax.broadcasted_iota(jnp.int32, (_B_T, 1), 0)
    mask = jnp.logical_and(tok >= start, tok < end)
    contrib = jnp.where(mask, dot - rs_ref[...] * zpsc_ref[0, 0][None, :], 0.0)

    @pl.when(first)
    def _():
        out_ref[...] = contrib

    @pl.when(jnp.logical_not(first))
    def _():
        out_ref[...] = out_ref[...] + contrib


def _permute_body(x_ref, p_ref, o_ref, rs_ref):
    x = x_ref[...]
    o_ref[...] = jax.lax.dot_general(
        x.astype(jnp.bfloat16), p_ref[...],
        (((1,), (0,)), ((), ())), preferred_element_type=jnp.float32,
    ).astype(jnp.bfloat16)
    rs_ref[...] = jnp.sum(x, axis=1, keepdims=True)


def _permute_x(inputs):
    col = jnp.arange(_HIDDEN, dtype=jnp.int32)
    src = jnp.where(col < _PACKED, col * 2, (col - _PACKED) * 2 + 1)
    perm = (col[:, None] == src[None, :]).astype(jnp.bfloat16)
    return pl.pallas_call(
        _permute_body,
        grid=(_T_BLOCKS,),
        in_specs=[
            pl.BlockSpec((_B_T, _HIDDEN), lambda i: (i, 0)),
            pl.BlockSpec((_HIDDEN, _HIDDEN), lambda i: (0, 0)),
        ],
        out_specs=[
            pl.BlockSpec((_B_T, _HIDDEN), lambda i: (i, 0)),
            pl.BlockSpec((_B_T, 1), lambda i: (i, 0)),
        ],
        out_shape=[
            jax.ShapeDtypeStruct((_TOKENS, _HIDDEN), jnp.bfloat16),
            jax.ShapeDtypeStruct((_TOKENS, 1), jnp.float32),
        ],
    )(inputs, perm)


def _build_plan(input_offsets, tokens_per_expert):
    starts = input_offsets.astype(jnp.int32)
    ends = starts + tokens_per_expert.astype(jnp.int32)
    blk_lo = jnp.arange(_T_BLOCKS, dtype=jnp.int32) * _B_T
    ov = jnp.logical_and(
        starts[None, :] < blk_lo[:, None] + _B_T, ends[None, :] > blk_lo[:, None]
    )
    flat = ov.reshape(-1)
    order = jnp.argsort(jnp.logical_not(flat), stable=True)[:_G]
    valid = flat[order]
    blk_ids = (order // _NUM_EXPERTS).astype(jnp.int32)
    e_ids = (order % _NUM_EXPERTS).astype(jnp.int32)
    n_real = jnp.sum(flat.astype(jnp.int32))
    last = jnp.maximum(n_real - 1, 0)
    blk_ids = jnp.where(valid, blk_ids, blk_ids[last])
    e_ids = jnp.where(valid, e_ids, e_ids[last])
    st = jnp.where(valid, starts[e_ids], 0)
    en = jnp.where(valid, ends[e_ids], 0)
    return jnp.stack([blk_ids, st, en, e_ids]).astype(jnp.int32)


def kernel(inputs, packed_weights, scales, zero_points, expert_ids,
           tokens_per_expert, input_offsets):
    del expert_ids
    meta = _build_plan(input_offsets, tokens_per_expert)
    x_perm, rowsums = _permute_x(inputs)
    sc3 = scales.reshape(_NUM_EXPERTS * _F_BLOCKS, 1, _B_F)
    zpsc3 = (zero_points * scales).reshape(_NUM_EXPERTS * _F_BLOCKS, 1, _B_F)

    grid_spec = pltpu.PrefetchScalarGridSpec(
        num_scalar_prefetch=1,
        grid=(_F_BLOCKS, _G),
        in_specs=[
            pl.BlockSpec((_B_T, _HIDDEN), lambda f, i, m: (m[0, i], 0)),
            pl.BlockSpec((1, _B_F, _PACKED), lambda f, i, m: (m[3, i], f, 0)),
            pl.BlockSpec((1, 1, _B_F), lambda f, i, m: (m[3, i] * _F_BLOCKS + f, 0, 0)),
            pl.BlockSpec((1, 1, _B_F), lambda f, i, m: (m[3, i] * _F_BLOCKS + f, 0, 0)),
            pl.BlockSpec((_B_T, 1), lambda f, i, m: (m[0, i], 0)),
        ],
        out_specs=pl.BlockSpec((_B_T, _B_F), lambda f, i, m: (m[0, i], f)),
        scratch_shapes=[pltpu.VMEM((_B_F, _HIDDEN), jnp.bfloat16)],
    )
    out = pl.pallas_call(
        _moe_body,
        grid_spec=grid_spec,
        out_shape=jax.ShapeDtypeStruct((_TOKENS, _FFN), jnp.float32),
        compiler_params=pltpu.CompilerParams(
            dimension_semantics=("arbitrary", "arbitrary"),
        ),
    )(meta, x_perm, packed_weights, sc3, zpsc3, rowsums)
    return out
